# Initial kernel scaffold; baseline (speedup 1.0000x reference)
import jax
import jax.numpy as jnp
from jax.experimental import pallas as pl


def kernel(x, edge_index, edge_attr, batch, W_in, b_in, W0s, b0s, W1s, b1s, W_pred, b_pred):
    raise NotImplementedError("write your pallas kernel here")


# speedup vs baseline: 3.5161x; 3.5161x over previous
import functools

import jax
import jax.numpy as jnp
from jax import lax
from jax.experimental import pallas as pl
from jax.experimental.pallas import tpu as pltpu
from jax.experimental.pallas import tpu_sc as plsc

N = 50000
E = 800000
NODE_DIM = 128
EMB = 128
EDGE_DIM = 16
L = 4
G = 64

NC = 2
NS = 16
NW = NC * NS
CH = 128
NCHW = 200
EP = NW * NCHW * CH
SLAB_CH = 8
NSLAB = NCHW // SLAB_CH
SLAB_E = SLAB_CH * CH
NP = 51200
RPT = NP // NS
ZR = 160

BN = 2000
GRID = N // BN


def _sc_body(dst_hbm, ea_hbm, out_hbm, data_v, idx_v, shared):
    cid = lax.axis_index("c")
    tid = lax.axis_index("s")
    wid = tid * NC + cid

    def zrow(i, _):
        data_v[i, :] = jnp.zeros((EDGE_DIM,), jnp.float32)
        return _
    lax.fori_loop(0, ZR, zrow, None)

    def zcopy(k, _):
        pltpu.sync_copy(data_v.at[pl.ds(0, ZR)],
                        shared.at[pl.ds(tid * RPT + k * ZR, ZR)])
        return _
    lax.fori_loop(0, RPT // ZR, zcopy, None)
    plsc.subcore_barrier()

    ch0 = wid * NCHW

    def slab(s, _):
        e0 = (ch0 + s * SLAB_CH) * CH
        pltpu.sync_copy(ea_hbm.at[pl.ds(e0, SLAB_E)], data_v)
        pltpu.sync_copy(dst_hbm.at[pl.ds(ch0 + s * SLAB_CH, SLAB_CH)], idx_v)

        def chunk(j, _):
            pltpu.sync_copy(data_v.at[pl.ds(j * CH, CH)],
                            shared.at[idx_v.at[j]],
                            add=True)
            return _
        lax.fori_loop(0, SLAB_CH, chunk, None)
        return _
    lax.fori_loop(0, NSLAB, slab, None)
    plsc.subcore_barrier()

    pltpu.sync_copy(shared.at[pl.ds(tid * RPT, RPT)],
                    out_hbm.at[cid, pl.ds(tid * RPT, RPT)])


@jax.jit
def _sc_edge_segsum(dst_p, ea_p):
    mesh = plsc.VectorSubcoreMesh(core_axis_name="c", subcore_axis_name="s",
                                  num_cores=NC, num_subcores=NS)
    return pl.kernel(
        _sc_body,
        out_type=jax.ShapeDtypeStruct((NC, NP, EDGE_DIM), jnp.float32),
        mesh=mesh,
        scratch_types=[
            pltpu.VMEM((SLAB_E, EDGE_DIM), jnp.float32),
            pltpu.VMEM((SLAB_CH, CH), jnp.int32),
            pltpu.VMEM_SHARED((NP, EDGE_DIM), jnp.float32),
        ],
        compiler_params=pltpu.CompilerParams(use_tc_tiling_on_sc=False),
    )(dst_p, ea_p)


def _tc_body(x_ref, agg_ref, batch_ref, Win_ref, bin_ref, W0_ref, b0_ref,
             W1a_ref, W1b_ref, b1_ref, Wp_ref, bp_ref, out_ref, s_acc, c_acc):
    i = pl.program_id(0)
    f32 = jnp.float32
    dn = (((1,), (1,)), ((), ()))

    aggr = agg_ref[0] + agg_ref[1]
    h = lax.dot_general(x_ref[...], Win_ref[...], dn,
                        preferred_element_type=f32) + bin_ref[...]
    for l in range(L):
        t = jnp.maximum(
            lax.dot_general(h, W0_ref[l], dn, preferred_element_type=f32)
            + b0_ref[l], 0.0)
        u = (lax.dot_general(t, W1a_ref[l], dn, preferred_element_type=f32)
             + lax.dot_general(aggr, W1b_ref[l], dn, preferred_element_type=f32)
             + b1_ref[l])
        h = h + jnp.maximum(u, 0.0)

    p = lax.dot_general(h, Wp_ref[...], dn, preferred_element_type=f32)
    gids = lax.broadcasted_iota(jnp.int32, (1, G), 1)
    onehot = (batch_ref[0].reshape(BN, 1) == gids).astype(f32)
    sp = lax.dot_general(p, onehot, (((0,), (0,)), ((), ())),
                         preferred_element_type=f32)
    cp = jnp.sum(onehot, axis=0, keepdims=True)

    @pl.when(i == 0)
    def _():
        s_acc[...] = sp
        c_acc[...] = cp

    @pl.when(i > 0)
    def _():
        s_acc[...] += sp
        c_acc[...] += cp

    @pl.when(i == GRID - 1)
    def _():
        out_ref[...] = s_acc[...] / jnp.maximum(c_acc[...], 1.0) + bp_ref[...]


@jax.jit
def _tc_forward(x, agg, batch3, W_in, b_in2, W0s, b0s3, W1a, W1b, b1s3,
                W_pred, b_pred2):
    full = lambda shape: pl.BlockSpec(shape, lambda i: (0,) * len(shape))
    return pl.pallas_call(
        _tc_body,
        grid=(GRID,),
        in_specs=[
            pl.BlockSpec((BN, NODE_DIM), lambda i: (i, 0)),
            pl.BlockSpec((NC, BN, EDGE_DIM), lambda i: (0, i, 0)),
            pl.BlockSpec((1, 1, BN), lambda i: (i, 0, 0)),
            full((EMB, NODE_DIM)),
            full((1, EMB)),
            full((L, EDGE_DIM, EMB)),
            full((L, 1, EDGE_DIM)),
            full((L, EMB, EDGE_DIM)),
            full((L, EMB, EDGE_DIM)),
            full((L, 1, EMB)),
            full((1, EMB)),
            full((1, 1)),
        ],
        out_specs=pl.BlockSpec((1, G), lambda i: (0, 0)),
        out_shape=jax.ShapeDtypeStruct((1, G), jnp.float32),
        scratch_shapes=[
            pltpu.VMEM((1, G), jnp.float32),
            pltpu.VMEM((1, G), jnp.float32),
        ],
    )(x, agg, batch3, W_in, b_in2, W0s, b0s3, W1a, W1b, b1s3, W_pred, b_pred2)


def kernel(x, edge_index, edge_attr, batch, W_in, b_in, W0s, b0s, W1s, b1s,
           W_pred, b_pred):
    dst = edge_index[1]
    pad = EP - E
    ea_p = jnp.concatenate(
        [edge_attr, jnp.zeros((pad, EDGE_DIM), edge_attr.dtype)], axis=0)
    dst_p = jnp.concatenate(
        [dst, jnp.zeros((pad,), dst.dtype)], axis=0).reshape(EP // CH, CH)

    agg = _sc_edge_segsum(dst_p, ea_p)

    out = _tc_forward(
        x, agg, batch.reshape(GRID, 1, BN),
        W_in, b_in.reshape(1, EMB),
        W0s, b0s.reshape(L, 1, EDGE_DIM),
        W1s[:, :, :EDGE_DIM], W1s[:, :, EDGE_DIM:],
        b1s.reshape(L, 1, EMB),
        W_pred, b_pred.reshape(1, 1),
    )
    return out.reshape(-1)


# speedup vs baseline: 5.2500x; 1.4931x over previous
import functools

import jax
import jax.numpy as jnp
from jax import lax
from jax.experimental import pallas as pl
from jax.experimental.pallas import tpu as pltpu
from jax.experimental.pallas import tpu_sc as plsc

N = 50000
E = 800000
NODE_DIM = 128
EMB = 128
EDGE_DIM = 16
L = 4
G = 64

NC = 2
NS = 16
NW = NC * NS
CH = 128
NCH = E // CH
NCHW = NCH // NW
NW_X = NCH - NCHW * NW
SLAB_CH = 8
NSLAB = NCHW // SLAB_CH
SLAB_E = SLAB_CH * CH
NP = 51200
RPT = NP // NS
ZR = 160

BN = 2000
GRID = N // BN


def _sc_body(dst_hbm, ea_hbm, out_hbm, data_v, idx_v, shared):
    cid = lax.axis_index("c")
    tid = lax.axis_index("s")
    wid = tid * NC + cid

    def zrow(i, _):
        data_v[i, :] = jnp.zeros((EDGE_DIM,), jnp.float32)
        return _
    lax.fori_loop(0, ZR, zrow, None)

    def zcopy(k, _):
        pltpu.sync_copy(data_v.at[pl.ds(0, ZR)],
                        shared.at[pl.ds(tid * RPT + k * ZR, ZR)])
        return _
    lax.fori_loop(0, RPT // ZR, zcopy, None)
    plsc.subcore_barrier()

    ch0 = wid * NCHW + jnp.minimum(wid, NW_X)
    rem = jnp.where(wid < NW_X, NCHW + 1 - NSLAB * SLAB_CH,
                    NCHW - NSLAB * SLAB_CH)

    def slab(s, _):
        c0 = ch0 + s * SLAB_CH
        pltpu.sync_copy(ea_hbm.at[pl.ds(c0 * CH, SLAB_E)], data_v)
        pltpu.sync_copy(dst_hbm.at[pl.ds(c0, SLAB_CH)], idx_v)

        def chunk(j, _):
            pltpu.sync_copy(data_v.at[pl.ds(j * CH, CH)],
                            shared.at[idx_v.at[j]],
                            add=True)
            return _
        lax.fori_loop(0, SLAB_CH, chunk, None)
        return _
    lax.fori_loop(0, NSLAB, slab, None)

    def tail(t, _):
        c0 = ch0 + NSLAB * SLAB_CH + t
        pltpu.sync_copy(ea_hbm.at[pl.ds(c0 * CH, CH)],
                        data_v.at[pl.ds(0, CH)])
        pltpu.sync_copy(dst_hbm.at[pl.ds(c0, 1)], idx_v.at[pl.ds(0, 1)])
        pltpu.sync_copy(data_v.at[pl.ds(0, CH)],
                        shared.at[idx_v.at[0]],
                        add=True)
        return _
    lax.fori_loop(0, rem, tail, None)
    plsc.subcore_barrier()

    pltpu.sync_copy(shared.at[pl.ds(tid * RPT, RPT)],
                    out_hbm.at[cid, pl.ds(tid * RPT, RPT)])


@jax.jit
def _sc_edge_segsum(dst_p, ea_p):
    mesh = plsc.VectorSubcoreMesh(core_axis_name="c", subcore_axis_name="s",
                                  num_cores=NC, num_subcores=NS)
    return pl.kernel(
        _sc_body,
        out_type=jax.ShapeDtypeStruct((NC, NP, EDGE_DIM), jnp.float32),
        mesh=mesh,
        scratch_types=[
            pltpu.VMEM((SLAB_E, EDGE_DIM), jnp.float32),
            pltpu.VMEM((SLAB_CH, CH), jnp.int32),
            pltpu.VMEM_SHARED((NP, EDGE_DIM), jnp.float32),
        ],
        compiler_params=pltpu.CompilerParams(use_tc_tiling_on_sc=False),
    )(dst_p, ea_p)


def _tc_body(x_ref, agg_ref, batch_ref, Win_ref, bin_ref, W0_ref, b0_ref,
             W1a_ref, W1b_ref, b1_ref, Wp_ref, bp_ref, out_ref, s_acc, c_acc):
    i = pl.program_id(0)
    f32 = jnp.float32
    dn = (((1,), (1,)), ((), ()))
    dot = functools.partial(lax.dot_general, dimension_numbers=dn,
                            preferred_element_type=f32)

    aggr = agg_ref[0] + agg_ref[1]
    h = dot(x_ref[...], Win_ref[...]) + bin_ref[...]
    for l in range(L):
        t = jnp.maximum(dot(h, W0_ref[l]) + b0_ref[l], 0.0)
        u = (dot(t, W1a_ref[l]) + dot(aggr, W1b_ref[l])
             + b1_ref[l])
        h = h + jnp.maximum(u, 0.0)

    gids = lax.broadcasted_iota(jnp.int32, (1, G), 1)
    onehot = (batch_ref[0].reshape(BN, 1) == gids).astype(f32)
    h_hi = h.astype(jnp.bfloat16).astype(f32)
    h_lo = h - h_hi
    pdn = (((0,), (0,)), ((), ()))
    sp = (lax.dot_general(onehot, h_hi, pdn, preferred_element_type=f32)
          + lax.dot_general(onehot, h_lo, pdn, preferred_element_type=f32))
    cp = jnp.sum(onehot, axis=0, keepdims=True)

    @pl.when(i == 0)
    def _():
        s_acc[...] = sp
        c_acc[...] = cp

    @pl.when(i > 0)
    def _():
        s_acc[...] += sp
        c_acc[...] += cp

    @pl.when(i == GRID - 1)
    def _():
        hg = s_acc[...] / jnp.maximum(c_acc[...], 1.0).reshape(G, 1)
        out_ref[...] = (lax.dot_general(
            Wp_ref[...], hg, dn, preferred_element_type=f32)
            + bp_ref[...])


@jax.jit
def _tc_forward(x, agg, batch3, W_in, b_in2, W0s, b0s3, W1a, W1b, b1s3,
                W_pred, b_pred2):
    full = lambda shape: pl.BlockSpec(shape, lambda i: (0,) * len(shape))
    return pl.pallas_call(
        _tc_body,
        grid=(GRID,),
        in_specs=[
            pl.BlockSpec((BN, NODE_DIM), lambda i: (i, 0)),
            pl.BlockSpec((NC, BN, EDGE_DIM), lambda i: (0, i, 0)),
            pl.BlockSpec((1, 1, BN), lambda i: (i, 0, 0)),
            full((EMB, NODE_DIM)),
            full((1, EMB)),
            full((L, EDGE_DIM, EMB)),
            full((L, 1, EDGE_DIM)),
            full((L, EMB, EDGE_DIM)),
            full((L, EMB, EDGE_DIM)),
            full((L, 1, EMB)),
            full((1, EMB)),
            full((1, 1)),
        ],
        out_specs=pl.BlockSpec((1, G), lambda i: (0, 0)),
        out_shape=jax.ShapeDtypeStruct((1, G), jnp.float32),
        scratch_shapes=[
            pltpu.VMEM((G, EMB), jnp.float32),
            pltpu.VMEM((1, G), jnp.float32),
        ],
    )(x, agg, batch3, W_in, b_in2, W0s, b0s3, W1a, W1b, b1s3, W_pred, b_pred2)


def kernel(x, edge_index, edge_attr, batch, W_in, b_in, W0s, b0s, W1s, b1s,
           W_pred, b_pred):
    dst_r = edge_index[1].reshape(NCH, CH)

    agg = _sc_edge_segsum(dst_r, edge_attr)

    out = _tc_forward(
        x, agg, batch.reshape(GRID, 1, BN),
        W_in, b_in.reshape(1, EMB),
        W0s, b0s.reshape(L, 1, EDGE_DIM),
        W1s[:, :, :EDGE_DIM], W1s[:, :, EDGE_DIM:],
        b1s.reshape(L, 1, EMB),
        W_pred, b_pred.reshape(1, 1),
    )
    return out.reshape(-1)


# speedup vs baseline: 5.8070x; 1.1061x over previous
import functools

import jax
import jax.numpy as jnp
from jax import lax
from jax.experimental import pallas as pl
from jax.experimental.pallas import tpu as pltpu
from jax.experimental.pallas import tpu_sc as plsc

N = 50000
E = 800000
NODE_DIM = 128
EMB = 128
EDGE_DIM = 16
L = 4
G = 64

NC = 2
NS = 16
NW = NC * NS
CH = 128
NCH = E // CH
NCHW = NCH // NW
NW_X = NCH - NCHW * NW
SLAB_CH = 16
NSLAB = NCHW // SLAB_CH
SLAB_E = SLAB_CH * CH
NP = 51200
RPT = NP // NS
ZR = 160

BN = 10000
GRID = N // BN


def _sc_body(dst_hbm, ea_hbm, out_hbm, data_v, idx_v, shared, sem_st, sem_sc):
    cid = lax.axis_index("c")
    tid = lax.axis_index("s")
    wid = tid * NC + cid

    def zrow(i, _):
        data_v[0, i, :] = jnp.zeros((EDGE_DIM,), jnp.float32)
        return _
    lax.fori_loop(0, ZR, zrow, None)

    def zcopy(k, _):
        pltpu.sync_copy(data_v.at[0, pl.ds(0, ZR)],
                        shared.at[pl.ds(tid * RPT + k * ZR, ZR)])
        return _
    lax.fori_loop(0, RPT // ZR, zcopy, None)
    plsc.subcore_barrier()

    ch0 = wid * NCHW + jnp.minimum(wid, NW_X)
    rem = jnp.where(wid < NW_X, NCHW + 1 - NSLAB * SLAB_CH,
                    NCHW - NSLAB * SLAB_CH)

    def stage(s, buf):
        c0 = ch0 + s * SLAB_CH
        return (pltpu.async_copy(ea_hbm.at[pl.ds(c0 * CH, SLAB_E)],
                                 data_v.at[buf], sem_st),
                pltpu.async_copy(dst_hbm.at[1, pl.ds(c0, SLAB_CH)],
                                 idx_v.at[buf], sem_st))

    descs = stage(0, 0)
    for s in range(NSLAB):
        cur = s % 2
        descs[0].wait()
        descs[1].wait()
        if s + 1 < NSLAB:
            descs = stage(s + 1, (s + 1) % 2)
        scat = [pltpu.async_copy(data_v.at[cur, pl.ds(j * CH, CH)],
                                 shared.at[idx_v.at[cur, j]],
                                 sem_sc, add=True)
                for j in range(SLAB_CH)]
        for d in scat:
            d.wait()

    def tail(t, _):
        c0 = ch0 + NSLAB * SLAB_CH + t
        pltpu.sync_copy(ea_hbm.at[pl.ds(c0 * CH, CH)],
                        data_v.at[0, pl.ds(0, CH)])
        pltpu.sync_copy(dst_hbm.at[1, pl.ds(c0, 1)], idx_v.at[0, pl.ds(0, 1)])
        pltpu.sync_copy(data_v.at[0, pl.ds(0, CH)],
                        shared.at[idx_v.at[0, 0]],
                        add=True)
        return _
    lax.fori_loop(0, rem, tail, None)
    plsc.subcore_barrier()

    pltpu.sync_copy(shared.at[pl.ds(tid * RPT, RPT)],
                    out_hbm.at[cid, pl.ds(tid * RPT, RPT)])


@jax.jit
def _sc_edge_segsum(dst_p, ea_p):
    mesh = plsc.VectorSubcoreMesh(core_axis_name="c", subcore_axis_name="s",
                                  num_cores=NC, num_subcores=NS)
    return pl.kernel(
        _sc_body,
        out_type=jax.ShapeDtypeStruct((NC, NP, EDGE_DIM), jnp.float32),
        mesh=mesh,
        scratch_types=[
            pltpu.VMEM((2, SLAB_E, EDGE_DIM), jnp.float32),
            pltpu.VMEM((2, SLAB_CH, CH), jnp.int32),
            pltpu.VMEM_SHARED((NP, EDGE_DIM), jnp.float32),
            pltpu.SemaphoreType.DMA,
            pltpu.SemaphoreType.DMA,
        ],
        compiler_params=pltpu.CompilerParams(use_tc_tiling_on_sc=False),
    )(dst_p, ea_p)


def _tc_body(x_ref, agg_ref, batch_ref, Win_ref, bin_ref, W0_ref, b0_ref,
             W1a_ref, W1b_ref, b1_ref, Wp_ref, bp_ref, out_ref, s_acc, c_acc):
    i = pl.program_id(0)
    f32 = jnp.float32
    dn = (((1,), (1,)), ((), ()))
    dot = functools.partial(lax.dot_general, dimension_numbers=dn,
                            preferred_element_type=f32)

    aggr = agg_ref[0] + agg_ref[1]
    h = dot(x_ref[...], Win_ref[...]) + bin_ref[...]
    for l in range(L):
        t = jnp.maximum(dot(h, W0_ref[l]) + b0_ref[l], 0.0)
        u = (dot(t, W1a_ref[l]) + dot(aggr, W1b_ref[l])
             + b1_ref[l])
        h = h + jnp.maximum(u, 0.0)

    gids = lax.broadcasted_iota(jnp.int32, (1, G), 1)
    onehot = (batch_ref[0].reshape(BN, 1) == gids).astype(f32)
    h_hi = h.astype(jnp.bfloat16).astype(f32)
    h_lo = h - h_hi
    pdn = (((0,), (0,)), ((), ()))
    sp = (lax.dot_general(onehot, h_hi, pdn, preferred_element_type=f32)
          + lax.dot_general(onehot, h_lo, pdn, preferred_element_type=f32))
    cp = jnp.sum(onehot, axis=0, keepdims=True)

    @pl.when(i == 0)
    def _():
        s_acc[...] = sp
        c_acc[...] = cp

    @pl.when(i > 0)
    def _():
        s_acc[...] += sp
        c_acc[...] += cp

    @pl.when(i == GRID - 1)
    def _():
        hg = s_acc[...] / jnp.maximum(c_acc[...], 1.0).reshape(G, 1)
        out_ref[...] = (lax.dot_general(
            Wp_ref[...], hg, dn, preferred_element_type=f32)
            + bp_ref[...])


@jax.jit
def _tc_forward(x, agg, batch3, W_in, b_in2, W0s, b0s3, W1a, W1b, b1s3,
                W_pred, b_pred2):
    full = lambda shape: pl.BlockSpec(shape, lambda i: (0,) * len(shape))
    return pl.pallas_call(
        _tc_body,
        grid=(GRID,),
        in_specs=[
            pl.BlockSpec((BN, NODE_DIM), lambda i: (i, 0)),
            pl.BlockSpec((NC, BN, EDGE_DIM), lambda i: (0, i, 0)),
            pl.BlockSpec((1, 1, BN), lambda i: (i, 0, 0)),
            full((EMB, NODE_DIM)),
            full((1, EMB)),
            full((L, EDGE_DIM, EMB)),
            full((L, 1, EDGE_DIM)),
            full((L, EMB, EDGE_DIM)),
            full((L, EMB, EDGE_DIM)),
            full((L, 1, EMB)),
            full((1, EMB)),
            full((1, 1)),
        ],
        out_specs=pl.BlockSpec((1, G), lambda i: (0, 0)),
        out_shape=jax.ShapeDtypeStruct((1, G), jnp.float32),
        scratch_shapes=[
            pltpu.VMEM((G, EMB), jnp.float32),
            pltpu.VMEM((1, G), jnp.float32),
        ],
    )(x, agg, batch3, W_in, b_in2, W0s, b0s3, W1a, W1b, b1s3, W_pred, b_pred2)


def kernel(x, edge_index, edge_attr, batch, W_in, b_in, W0s, b0s, W1s, b1s,
           W_pred, b_pred):
    ei_r = edge_index.reshape(2, NCH, CH)

    agg = _sc_edge_segsum(ei_r, edge_attr)

    out = _tc_forward(
        x, agg, batch.reshape(GRID, 1, BN),
        W_in, b_in.reshape(1, EMB),
        W0s, b0s.reshape(L, 1, EDGE_DIM),
        W1s[:, :, :EDGE_DIM], W1s[:, :, EDGE_DIM:],
        b1s.reshape(L, 1, EMB),
        W_pred, b_pred.reshape(1, 1),
    )
    return out.reshape(-1)


# speedup vs baseline: 5.8156x; 1.0015x over previous
import functools

import jax
import jax.numpy as jnp
from jax import lax
from jax.experimental import pallas as pl
from jax.experimental.pallas import tpu as pltpu
from jax.experimental.pallas import tpu_sc as plsc

N = 50000
E = 800000
NODE_DIM = 128
EMB = 128
EDGE_DIM = 16
L = 4
G = 64

NC = 2
NS = 16
NW = NC * NS
CH = 128
NCH = E // CH
NCHW = NCH // NW
NW_X = NCH - NCHW * NW
SLAB_CH = 16
NSLAB = NCHW // SLAB_CH
SLAB_E = SLAB_CH * CH
NP = 51200
RPT = NP // NS
ZR = 160

BN = 10000
GRID = N // BN


def _sc_body(dst_hbm, ea_hbm, out_hbm, data_v, idx_v, shared, sem_st, sem_sc):
    cid = lax.axis_index("c")
    tid = lax.axis_index("s")
    wid = tid * NC + cid

    def zrow(i, _):
        data_v[0, i, :] = jnp.zeros((EDGE_DIM,), jnp.float32)
        return _
    lax.fori_loop(0, ZR, zrow, None)

    def zcopy(k, _):
        pltpu.sync_copy(data_v.at[0, pl.ds(0, ZR)],
                        shared.at[pl.ds(tid * RPT + k * ZR, ZR)])
        return _
    lax.fori_loop(0, RPT // ZR, zcopy, None)
    plsc.subcore_barrier()

    ch0 = wid * NCHW + jnp.minimum(wid, NW_X)
    rem = jnp.where(wid < NW_X, NCHW + 1 - NSLAB * SLAB_CH,
                    NCHW - NSLAB * SLAB_CH)

    def stage(s, buf):
        c0 = ch0 + s * SLAB_CH
        return (pltpu.async_copy(ea_hbm.at[pl.ds(c0 * CH, SLAB_E)],
                                 data_v.at[buf], sem_st),
                pltpu.async_copy(dst_hbm.at[1, pl.ds(c0 * CH, SLAB_E)],
                                 idx_v.at[buf], sem_st))

    descs = stage(0, 0)
    for s in range(NSLAB):
        cur = s % 2
        descs[0].wait()
        descs[1].wait()
        if s + 1 < NSLAB:
            descs = stage(s + 1, (s + 1) % 2)
        scat = [pltpu.async_copy(data_v.at[cur, pl.ds(j * CH, CH)],
                                 shared.at[idx_v.at[cur, pl.ds(j * CH, CH)]],
                                 sem_sc, add=True)
                for j in range(SLAB_CH)]
        for d in scat:
            d.wait()

    def tail(t, _):
        c0 = ch0 + NSLAB * SLAB_CH + t
        pltpu.sync_copy(ea_hbm.at[pl.ds(c0 * CH, CH)],
                        data_v.at[0, pl.ds(0, CH)])
        pltpu.sync_copy(dst_hbm.at[1, pl.ds(c0 * CH, CH)],
                        idx_v.at[0, pl.ds(0, CH)])
        pltpu.sync_copy(data_v.at[0, pl.ds(0, CH)],
                        shared.at[idx_v.at[0, pl.ds(0, CH)]],
                        add=True)
        return _
    lax.fori_loop(0, rem, tail, None)
    plsc.subcore_barrier()

    pltpu.sync_copy(shared.at[pl.ds(tid * RPT, RPT)],
                    out_hbm.at[cid, pl.ds(tid * RPT, RPT)])


@jax.jit
def _sc_edge_segsum(dst_p, ea_p):
    mesh = plsc.VectorSubcoreMesh(core_axis_name="c", subcore_axis_name="s",
                                  num_cores=NC, num_subcores=NS)
    return pl.kernel(
        _sc_body,
        out_type=jax.ShapeDtypeStruct((NC, NP, EDGE_DIM), jnp.float32),
        mesh=mesh,
        scratch_types=[
            pltpu.VMEM((2, SLAB_E, EDGE_DIM), jnp.float32),
            pltpu.VMEM((2, SLAB_E), jnp.int32),
            pltpu.VMEM_SHARED((NP, EDGE_DIM), jnp.float32),
            pltpu.SemaphoreType.DMA,
            pltpu.SemaphoreType.DMA,
        ],
        compiler_params=pltpu.CompilerParams(use_tc_tiling_on_sc=False),
    )(dst_p, ea_p)


def _tc_body(x_ref, agg_ref, batch_ref, Win_ref, bin_ref, W0_ref, b0_ref,
             W1a_ref, W1b_ref, b1_ref, Wp_ref, bp_ref, out_ref, s_acc, c_acc):
    i = pl.program_id(0)
    f32 = jnp.float32
    dn = (((1,), (1,)), ((), ()))
    dot = functools.partial(lax.dot_general, dimension_numbers=dn,
                            preferred_element_type=f32)

    aggr = agg_ref[0] + agg_ref[1]
    h = dot(x_ref[...], Win_ref[...]) + bin_ref[...]
    for l in range(L):
        t = jnp.maximum(dot(h, W0_ref[l]) + b0_ref[l], 0.0)
        u = (dot(t, W1a_ref[l]) + dot(aggr, W1b_ref[l])
             + b1_ref[l])
        h = h + jnp.maximum(u, 0.0)

    gids = lax.broadcasted_iota(jnp.int32, (1, G), 1)
    onehot = (batch_ref[0].reshape(BN, 1) == gids).astype(f32)
    h_hi = h.astype(jnp.bfloat16).astype(f32)
    h_lo = h - h_hi
    pdn = (((0,), (0,)), ((), ()))
    sp = (lax.dot_general(onehot, h_hi, pdn, preferred_element_type=f32)
          + lax.dot_general(onehot, h_lo, pdn, preferred_element_type=f32))
    cp = jnp.sum(onehot, axis=0, keepdims=True)

    @pl.when(i == 0)
    def _():
        s_acc[...] = sp
        c_acc[...] = cp

    @pl.when(i > 0)
    def _():
        s_acc[...] += sp
        c_acc[...] += cp

    @pl.when(i == GRID - 1)
    def _():
        hg = s_acc[...] / jnp.maximum(c_acc[...], 1.0).reshape(G, 1)
        out_ref[...] = (lax.dot_general(
            Wp_ref[...], hg, dn, preferred_element_type=f32)
            + bp_ref[...])


@jax.jit
def _tc_forward(x, agg, batch3, W_in, b_in2, W0s, b0s3, W1a, W1b, b1s3,
                W_pred, b_pred2):
    full = lambda shape: pl.BlockSpec(shape, lambda i: (0,) * len(shape))
    return pl.pallas_call(
        _tc_body,
        grid=(GRID,),
        in_specs=[
            pl.BlockSpec((BN, NODE_DIM), lambda i: (i, 0)),
            pl.BlockSpec((NC, BN, EDGE_DIM), lambda i: (0, i, 0)),
            pl.BlockSpec((1, 1, BN), lambda i: (i, 0, 0)),
            full((EMB, NODE_DIM)),
            full((1, EMB)),
            full((L, EDGE_DIM, EMB)),
            full((L, 1, EDGE_DIM)),
            full((L, EMB, EDGE_DIM)),
            full((L, EMB, EDGE_DIM)),
            full((L, 1, EMB)),
            full((1, EMB)),
            full((1, 1)),
        ],
        out_specs=pl.BlockSpec((1, G), lambda i: (0, 0)),
        out_shape=jax.ShapeDtypeStruct((1, G), jnp.float32),
        scratch_shapes=[
            pltpu.VMEM((G, EMB), jnp.float32),
            pltpu.VMEM((1, G), jnp.float32),
        ],
    )(x, agg, batch3, W_in, b_in2, W0s, b0s3, W1a, W1b, b1s3, W_pred, b_pred2)


def kernel(x, edge_index, edge_attr, batch, W_in, b_in, W0s, b0s, W1s, b1s,
           W_pred, b_pred):
    agg = _sc_edge_segsum(edge_index, edge_attr)

    out = _tc_forward(
        x, agg, batch.reshape(GRID, 1, BN),
        W_in, b_in.reshape(1, EMB),
        W0s, b0s.reshape(L, 1, EDGE_DIM),
        W1s[:, :, :EDGE_DIM], W1s[:, :, EDGE_DIM:],
        b1s.reshape(L, 1, EMB),
        W_pred, b_pred.reshape(1, 1),
    )
    return out.reshape(-1)


# speedup vs baseline: 5.8195x; 1.0007x over previous
import functools

import jax
import jax.numpy as jnp
from jax import lax
from jax.experimental import pallas as pl
from jax.experimental.pallas import tpu as pltpu
from jax.experimental.pallas import tpu_sc as plsc

N = 50000
E = 800000
NODE_DIM = 128
EMB = 128
EDGE_DIM = 16
L = 4
G = 64

NC = 2
NS = 16
NW = NC * NS
CH = 128
NCH = E // CH
NCHW = NCH // NW
NW_X = NCH - NCHW * NW
SLAB_CH = 16
NSLAB = NCHW // SLAB_CH
SLAB_E = SLAB_CH * CH
NP = 51200
RPT = NP // NS
ZR = 160

BN = 10000
GRID = N // BN


def _sc_body(dst_hbm, ea_hbm, out_hbm, data_v, idx_v, shared, sem_st, sem_sc):
    cid = lax.axis_index("c")
    tid = lax.axis_index("s")
    wid = tid * NC + cid

    def zrow(i, _):
        data_v[0, i, :] = jnp.zeros((EDGE_DIM,), jnp.float32)
        return _
    lax.fori_loop(0, ZR, zrow, None)

    def zcopy(k, _):
        pltpu.sync_copy(data_v.at[0, pl.ds(0, ZR)],
                        shared.at[pl.ds(tid * RPT + k * ZR, ZR)])
        return _
    lax.fori_loop(0, RPT // ZR, zcopy, None)
    plsc.subcore_barrier()

    ch0 = wid * NCHW + jnp.minimum(wid, NW_X)
    rem = jnp.where(wid < NW_X, NCHW + 1 - NSLAB * SLAB_CH,
                    NCHW - NSLAB * SLAB_CH)

    def stage(s, buf):
        c0 = ch0 + s * SLAB_CH
        return (pltpu.async_copy(ea_hbm.at[pl.ds(c0 * CH, SLAB_E)],
                                 data_v.at[buf], sem_st),
                pltpu.async_copy(dst_hbm.at[pl.ds(c0 * CH, SLAB_E)],
                                 idx_v.at[buf], sem_st))

    descs = stage(0, 0)
    for s in range(NSLAB):
        cur = s % 2
        descs[0].wait()
        descs[1].wait()
        if s + 1 < NSLAB:
            descs = stage(s + 1, (s + 1) % 2)
        scat = [pltpu.async_copy(data_v.at[cur, pl.ds(j * CH, CH)],
                                 shared.at[idx_v.at[cur, pl.ds(j * CH, CH)]],
                                 sem_sc, add=True)
                for j in range(SLAB_CH)]
        for d in scat:
            d.wait()

    def tail(t, _):
        c0 = ch0 + NSLAB * SLAB_CH + t
        pltpu.sync_copy(ea_hbm.at[pl.ds(c0 * CH, CH)],
                        data_v.at[0, pl.ds(0, CH)])
        pltpu.sync_copy(dst_hbm.at[pl.ds(c0 * CH, CH)],
                        idx_v.at[0, pl.ds(0, CH)])
        pltpu.sync_copy(data_v.at[0, pl.ds(0, CH)],
                        shared.at[idx_v.at[0, pl.ds(0, CH)]],
                        add=True)
        return _
    lax.fori_loop(0, rem, tail, None)
    plsc.subcore_barrier()

    pltpu.sync_copy(shared.at[pl.ds(tid * RPT, RPT)],
                    out_hbm.at[cid, pl.ds(tid * RPT, RPT)])


@jax.jit
def _sc_edge_segsum(dst_p, ea_p):
    mesh = plsc.VectorSubcoreMesh(core_axis_name="c", subcore_axis_name="s",
                                  num_cores=NC, num_subcores=NS)
    return pl.kernel(
        _sc_body,
        out_type=jax.ShapeDtypeStruct((NC, NP, EDGE_DIM), jnp.float32),
        mesh=mesh,
        scratch_types=[
            pltpu.VMEM((2, SLAB_E, EDGE_DIM), jnp.float32),
            pltpu.VMEM((2, SLAB_E), jnp.int32),
            pltpu.VMEM_SHARED((NP, EDGE_DIM), jnp.float32),
            pltpu.SemaphoreType.DMA,
            pltpu.SemaphoreType.DMA,
        ],
        compiler_params=pltpu.CompilerParams(use_tc_tiling_on_sc=False),
    )(dst_p, ea_p)


DBE = 80000


def _dst_body(ei_ref, out_ref):
    i = pl.program_id(0)
    out_ref[pl.ds(i * DBE, DBE)] = ei_ref[1]


@jax.jit
def _dst_extract(edge_index):
    return pl.pallas_call(
        _dst_body,
        grid=(E // DBE,),
        in_specs=[pl.BlockSpec((2, DBE), lambda i: (0, i))],
        out_specs=pl.BlockSpec((E,), lambda i: (0,)),
        out_shape=jax.ShapeDtypeStruct((E,), jnp.int32),
    )(edge_index)


def _tc_body(x_ref, agg_ref, batch_ref, Win_ref, bin_ref, W0_ref, b0_ref,
             W1a_ref, W1b_ref, b1_ref, Wp_ref, bp_ref, out_ref, s_acc, c_acc):
    i = pl.program_id(0)
    f32 = jnp.float32
    dn = (((1,), (1,)), ((), ()))
    dot = functools.partial(lax.dot_general, dimension_numbers=dn,
                            preferred_element_type=f32)

    aggr = agg_ref[0] + agg_ref[1]
    h = dot(x_ref[...], Win_ref[...]) + bin_ref[...]
    for l in range(L):
        t = jnp.maximum(dot(h, W0_ref[l]) + b0_ref[l], 0.0)
        u = (dot(t, W1a_ref[l]) + dot(aggr, W1b_ref[l])
             + b1_ref[l])
        h = h + jnp.maximum(u, 0.0)

    gids = lax.broadcasted_iota(jnp.int32, (1, G), 1)
    onehot = (batch_ref[0].reshape(BN, 1) == gids).astype(f32)
    h_hi = h.astype(jnp.bfloat16).astype(f32)
    h_lo = h - h_hi
    pdn = (((0,), (0,)), ((), ()))
    sp = (lax.dot_general(onehot, h_hi, pdn, preferred_element_type=f32)
          + lax.dot_general(onehot, h_lo, pdn, preferred_element_type=f32))
    cp = jnp.sum(onehot, axis=0, keepdims=True)

    @pl.when(i == 0)
    def _():
        s_acc[...] = sp
        c_acc[...] = cp

    @pl.when(i > 0)
    def _():
        s_acc[...] += sp
        c_acc[...] += cp

    @pl.when(i == GRID - 1)
    def _():
        hg = s_acc[...] / jnp.maximum(c_acc[...], 1.0).reshape(G, 1)
        out_ref[...] = (lax.dot_general(
            Wp_ref[...], hg, dn, preferred_element_type=f32)
            + bp_ref[...])


@jax.jit
def _tc_forward(x, agg, batch3, W_in, b_in2, W0s, b0s3, W1a, W1b, b1s3,
                W_pred, b_pred2):
    full = lambda shape: pl.BlockSpec(shape, lambda i: (0,) * len(shape))
    return pl.pallas_call(
        _tc_body,
        grid=(GRID,),
        in_specs=[
            pl.BlockSpec((BN, NODE_DIM), lambda i: (i, 0)),
            pl.BlockSpec((NC, BN, EDGE_DIM), lambda i: (0, i, 0)),
            pl.BlockSpec((1, 1, BN), lambda i: (i, 0, 0)),
            full((EMB, NODE_DIM)),
            full((1, EMB)),
            full((L, EDGE_DIM, EMB)),
            full((L, 1, EDGE_DIM)),
            full((L, EMB, EDGE_DIM)),
            full((L, EMB, EDGE_DIM)),
            full((L, 1, EMB)),
            full((1, EMB)),
            full((1, 1)),
        ],
        out_specs=pl.BlockSpec((1, G), lambda i: (0, 0)),
        out_shape=jax.ShapeDtypeStruct((1, G), jnp.float32),
        scratch_shapes=[
            pltpu.VMEM((G, EMB), jnp.float32),
            pltpu.VMEM((1, G), jnp.float32),
        ],
    )(x, agg, batch3, W_in, b_in2, W0s, b0s3, W1a, W1b, b1s3, W_pred, b_pred2)


def kernel(x, edge_index, edge_attr, batch, W_in, b_in, W0s, b0s, W1s, b1s,
           W_pred, b_pred):
    dst = _dst_extract(edge_index)

    agg = _sc_edge_segsum(dst, edge_attr)

    out = _tc_forward(
        x, agg, batch.reshape(GRID, 1, BN),
        W_in, b_in.reshape(1, EMB),
        W0s, b0s.reshape(L, 1, EDGE_DIM),
        W1s[:, :, :EDGE_DIM], W1s[:, :, EDGE_DIM:],
        b1s.reshape(L, 1, EMB),
        W_pred, b_pred.reshape(1, 1),
    )
    return out.reshape(-1)
